# baseline (speedup 1.0000x reference)
import jax
import jax.numpy as jnp
from jax.experimental import pallas as pl
from jax.experimental.pallas import tpu as pltpu

BATCH = 4096
N_CLUSTERS = 8192
R = 512
RG = 128
CK = 256


def _assign_kernel(x_ref, c_ref, out_ref):
    n_chunks = N_CLUSTERS // CK
    lanef = jax.lax.broadcasted_iota(jnp.int32, (RG, CK), 1).astype(jnp.float32)

    for g in range(R // RG):
        x0 = x_ref[pl.ds(g * RG, RG), 0:1]
        x1 = x_ref[pl.ds(g * RG, RG), 1:2]
        x0b = jnp.broadcast_to(x0, (RG, CK))
        x1b = jnp.broadcast_to(x1, (RG, CK))

        def body(t, carry, x0b=x0b, x1b=x1b):
            bestv, bidxf = carry
            c0 = c_ref[0:1, pl.ds(t * CK, CK)]
            c1 = c_ref[1:2, pl.ds(t * CK, CK)]
            d0 = x0b - c0
            d1 = x1b - c1
            dist = d0 * d0 + d1 * d1
            mask = dist < bestv
            bestv = jnp.where(mask, dist, bestv)
            bidxf = jnp.where(mask, t.astype(jnp.float32), bidxf)
            return bestv, bidxf

        bestv0 = jnp.full((RG, CK), jnp.inf, dtype=jnp.float32)
        bidxf0 = jnp.zeros((RG, CK), dtype=jnp.float32)
        bestv, bidxf = jax.lax.fori_loop(0, n_chunks, body, (bestv0, bidxf0),
                                         unroll=4)

        m = jnp.min(bestv, axis=-1, keepdims=True)
        cand = jnp.where(bestv == m, bidxf * CK + lanef, float(N_CLUSTERS))
        idxf = jnp.min(cand, axis=-1, keepdims=True)
        out_ref[pl.ds(g * RG, RG), :] = idxf.astype(jnp.int32)


def kernel(inputs, cluster_centers):
    centers_t = cluster_centers.T
    grid = (BATCH // R,)
    out2d = pl.pallas_call(
        _assign_kernel,
        grid=grid,
        in_specs=[
            pl.BlockSpec((R, 2), lambda i: (i, 0)),
            pl.BlockSpec((2, N_CLUSTERS), lambda i: (0, 0)),
        ],
        out_specs=pl.BlockSpec((R, 1), lambda i: (i, 0)),
        out_shape=jax.ShapeDtypeStruct((BATCH, 1), jnp.int32),
        compiler_params=pltpu.CompilerParams(
            dimension_semantics=("parallel",),
        ),
    )(inputs, centers_t)
    return out2d.reshape(BATCH)


# speedup vs baseline: 1.2932x; 1.2932x over previous
import jax
import jax.numpy as jnp
from jax.experimental import pallas as pl
from jax.experimental.pallas import tpu as pltpu

BATCH = 4096
N_CLUSTERS = 8192
R = 512
RG = 128
CK = 128


def _assign_kernel(x_ref, c_ref, out_ref):
    n_chunks = N_CLUSTERS // CK
    lanef = jax.lax.broadcasted_iota(jnp.int32, (RG, CK), 1).astype(jnp.float32)

    for g in range(R // RG):
        x0 = x_ref[pl.ds(g * RG, RG), 0:1]
        x1 = x_ref[pl.ds(g * RG, RG), 1:2]
        x0b = jnp.broadcast_to(x0, (RG, CK))
        x1b = jnp.broadcast_to(x1, (RG, CK))

        def body(t, carry, x0b=x0b, x1b=x1b):
            bestv, bidxf = carry
            c0 = c_ref[0:1, pl.ds(t * CK, CK)]
            c1 = c_ref[1:2, pl.ds(t * CK, CK)]
            d0 = x0b - c0
            d1 = x1b - c1
            dist = d0 * d0 + d1 * d1
            mask = dist < bestv
            bestv = jnp.where(mask, dist, bestv)
            bidxf = jnp.where(mask, t.astype(jnp.float32), bidxf)
            return bestv, bidxf

        bestv0 = jnp.full((RG, CK), jnp.inf, dtype=jnp.float32)
        bidxf0 = jnp.zeros((RG, CK), dtype=jnp.float32)
        bestv, bidxf = jax.lax.fori_loop(0, n_chunks, body, (bestv0, bidxf0),
                                         unroll=16)

        m = jnp.min(bestv, axis=-1, keepdims=True)
        cand = jnp.where(bestv == m, bidxf * CK + lanef, float(N_CLUSTERS))
        idxf = jnp.min(cand, axis=-1, keepdims=True)
        out_ref[pl.ds(g * RG, RG), :] = idxf.astype(jnp.int32)


def kernel(inputs, cluster_centers):
    centers_t = cluster_centers.T
    grid = (BATCH // R,)
    out2d = pl.pallas_call(
        _assign_kernel,
        grid=grid,
        in_specs=[
            pl.BlockSpec((R, 2), lambda i: (i, 0)),
            pl.BlockSpec((2, N_CLUSTERS), lambda i: (0, 0)),
        ],
        out_specs=pl.BlockSpec((R, 1), lambda i: (i, 0)),
        out_shape=jax.ShapeDtypeStruct((BATCH, 1), jnp.int32),
        compiler_params=pltpu.CompilerParams(
            dimension_semantics=("parallel",),
        ),
    )(inputs, centers_t)
    return out2d.reshape(BATCH)
